# Initial kernel scaffold; baseline (speedup 1.0000x reference)
import jax
import jax.numpy as jnp
from jax.experimental import pallas as pl


def kernel(x, edge_index, Wl1, Wr1, b1, Wl2, Wr2, b2, Wl3, Wr3, b3, fcW1, fcb1, fcW2, fcb2):
    raise NotImplementedError("write your pallas kernel here")


# speedup vs baseline: 6.8149x; 6.8149x over previous
import functools

import jax
import jax.numpy as jnp
from jax import lax
from jax.experimental import pallas as pl
from jax.experimental.pallas import tpu as pltpu
from jax.experimental.pallas import tpu_sc as plsc

_N = 10000
_NP = 10240
_E = 320000
_NC = 2
_NS = 16
_NW = _NC * _NS
_EPW = _E // _NW
_CH = 80
_NCHUNK = _EPW // _CH
_RPT = _NP // _NS
_DEGW = 16


def _make_edge_pass(D, with_deg):
    mesh = plsc.VectorSubcoreMesh(
        core_axis_name="c", subcore_axis_name="s",
        num_cores=_NC, num_subcores=_NS)
    scratch = [
        pltpu.VMEM((_CH,), jnp.int32),
        pltpu.VMEM((_CH,), jnp.int32),
        pltpu.VMEM((_CH, D), jnp.float32),
        pltpu.VMEM_SHARED((_NP, D), jnp.float32),
        pltpu.SemaphoreType.DMA,
    ]
    out_type = [jax.ShapeDtypeStruct((_NC, _NP, D), jnp.float32)]
    if with_deg:
        scratch += [
            pltpu.VMEM((_CH, _DEGW), jnp.float32),
            pltpu.VMEM_SHARED((_NP, _DEGW), jnp.float32),
        ]
        out_type.append(jax.ShapeDtypeStruct((_NC, _NP, _DEGW), jnp.float32))

    def body(*refs):
        if with_deg:
            (feat, srcs, dsts, zeros, ones, zeros16, out, degout,
             src_v, dst_v, rows_v, acc_sh, sem, ones_v, deg_sh) = refs
        else:
            (feat, srcs, dsts, zeros, out,
             src_v, dst_v, rows_v, acc_sh, sem) = refs
        c = lax.axis_index("c")
        s = lax.axis_index("s")
        wid = c * _NS + s
        r0 = s * _RPT
        pltpu.sync_copy(zeros.at[pl.ds(r0, _RPT)], acc_sh.at[pl.ds(r0, _RPT)])
        if with_deg:
            pltpu.sync_copy(zeros16.at[pl.ds(r0, _RPT)],
                            deg_sh.at[pl.ds(r0, _RPT)])
            pltpu.sync_copy(ones, ones_v)
        plsc.subcore_barrier()
        e0 = wid * _EPW

        def chunk(j, carry):
            off = e0 + j * _CH
            pltpu.sync_copy(srcs.at[pl.ds(off, _CH)], src_v)
            pltpu.sync_copy(dsts.at[pl.ds(off, _CH)], dst_v)
            pltpu.async_copy(feat.at[src_v], rows_v, sem).wait()
            pltpu.sync_copy(rows_v, acc_sh.at[dst_v], add=True)
            if with_deg:
                pltpu.sync_copy(ones_v, deg_sh.at[dst_v], add=True)
            return carry

        lax.fori_loop(0, _NCHUNK, chunk, 0)
        plsc.subcore_barrier()
        pltpu.sync_copy(acc_sh.at[pl.ds(r0, _RPT)],
                        out.at[c, pl.ds(r0, _RPT)])
        if with_deg:
            pltpu.sync_copy(deg_sh.at[pl.ds(r0, _RPT)],
                            degout.at[c, pl.ds(r0, _RPT)])

    return pl.kernel(body, out_type=tuple(out_type) if with_deg else out_type[0],
                     mesh=mesh, scratch_types=scratch,
                     compiler_params=pltpu.CompilerParams(
                         use_tc_tiling_on_sc=False))


_edge_pass_l1 = _make_edge_pass(128, True)
_edge_pass_l2 = _make_edge_pass(32, False)
_edge_pass_l3 = _make_edge_pass(16, False)

_BN = 1000


def _row_spec(w):
    return pl.BlockSpec((_BN, w), lambda i: (i, 0))


def _part_spec(core, w):
    return pl.BlockSpec((1, _BN, w), lambda i, c=core: (c, i, 0))


def _full_spec(shape):
    return pl.BlockSpec(shape, lambda i: tuple(0 for _ in shape))


def _inv_deg(d0, d1):
    deg = d0[0, :, 0:1] + d1[0, :, 0:1]
    return 1.0 / jnp.maximum(deg, 1.0)


def _tc1_body(p0, p1, d0, d1, xr, wl1, wr1, b1, wl2, wr2, p2o, r2o):
    inv = _inv_deg(d0, d1)
    agg = (p0[0] + p1[0]) * inv
    h = jnp.dot(agg, wl1[:], preferred_element_type=jnp.float32)
    h = h + jnp.dot(xr[:], wr1[:], preferred_element_type=jnp.float32)
    h = jnp.maximum(h + b1[:], 0.0)
    p2o[:] = jnp.dot(h, wl2[:], preferred_element_type=jnp.float32)
    r2o[:] = jnp.dot(h, wr2[:], preferred_element_type=jnp.float32)


def _tc1(parts, degp, x, Wl1, Wr1, b1, Wl2, Wr2):
    return pl.pallas_call(
        _tc1_body,
        grid=(_N // _BN,),
        in_specs=[_part_spec(0, 128), _part_spec(1, 128),
                  _part_spec(0, _DEGW), _part_spec(1, _DEGW),
                  _row_spec(128),
                  _full_spec((128, 256)), _full_spec((128, 256)),
                  _full_spec((1, 256)),
                  _full_spec((256, 32)), _full_spec((256, 32))],
        out_specs=[_row_spec(32), _row_spec(32)],
        out_shape=[jax.ShapeDtypeStruct((_N, 32), jnp.float32),
                   jax.ShapeDtypeStruct((_N, 32), jnp.float32)],
    )(parts, parts, degp, degp, x, Wl1, Wr1, b1, Wl2, Wr2)


def _tc2_body(q0, q1, d0, d1, r2, b2, wl3, wr3, p3o, r3o):
    inv = _inv_deg(d0, d1)
    h = jnp.maximum((q0[0] + q1[0]) * inv + b2[:] + r2[:], 0.0)
    p3o[:] = jnp.dot(h, wl3[:], preferred_element_type=jnp.float32)
    r3o[:] = jnp.dot(h, wr3[:], preferred_element_type=jnp.float32)


def _tc2(parts, degp, r2, b2, Wl3, Wr3):
    return pl.pallas_call(
        _tc2_body,
        grid=(_N // _BN,),
        in_specs=[_part_spec(0, 32), _part_spec(1, 32),
                  _part_spec(0, _DEGW), _part_spec(1, _DEGW),
                  _row_spec(32), _full_spec((1, 32)),
                  _full_spec((32, 16)), _full_spec((32, 16))],
        out_specs=[_row_spec(16), _row_spec(16)],
        out_shape=[jax.ShapeDtypeStruct((_N, 16), jnp.float32),
                   jax.ShapeDtypeStruct((_N, 16), jnp.float32)],
    )(parts, parts, degp, degp, r2, b2, Wl3, Wr3)


def _tc3_body(t0, t1, d0, d1, r3, b3, w1, bb1, w2, bb2, outo):
    inv = _inv_deg(d0, d1)
    h = jnp.maximum((t0[0] + t1[0]) * inv + b3[:] + r3[:], 0.0)
    h = jnp.maximum(jnp.dot(h, w1[:], preferred_element_type=jnp.float32)
                    + bb1[:], 0.0)
    logits = jnp.dot(h, w2[:], preferred_element_type=jnp.float32) + bb2[:]
    m = jnp.max(logits, axis=1, keepdims=True)
    z = logits - m
    lse = jnp.log(jnp.sum(jnp.exp(z), axis=1, keepdims=True))
    outo[:] = z - lse


def _tc3(parts, degp, r3, b3, fcW1, fcb1, fcW2, fcb2):
    return pl.pallas_call(
        _tc3_body,
        grid=(_N // _BN,),
        in_specs=[_part_spec(0, 16), _part_spec(1, 16),
                  _part_spec(0, _DEGW), _part_spec(1, _DEGW),
                  _row_spec(16), _full_spec((1, 16)),
                  _full_spec((16, 8)), _full_spec((1, 8)),
                  _full_spec((8, 2)), _full_spec((1, 2))],
        out_specs=[_row_spec(2)],
        out_shape=[jax.ShapeDtypeStruct((_N, 2), jnp.float32)],
    )(parts, parts, degp, degp, r3, b3, fcW1, fcb1, fcW2, fcb2)[0]


def kernel(x, edge_index, Wl1, Wr1, b1, Wl2, Wr2, b2, Wl3, Wr3, b3,
           fcW1, fcb1, fcW2, fcb2):
    src = edge_index[0].astype(jnp.int32)
    dst = edge_index[1].astype(jnp.int32)
    zeros128 = jnp.zeros((_NP, 128), jnp.float32)
    zeros32 = jnp.zeros((_NP, 32), jnp.float32)
    zeros16f = jnp.zeros((_NP, 16), jnp.float32)
    zerosdeg = jnp.zeros((_NP, _DEGW), jnp.float32)
    ones = jnp.ones((_CH, _DEGW), jnp.float32)

    agg1p, degp = _edge_pass_l1(x, src, dst, zeros128, ones, zerosdeg)
    p2, r2 = _tc1(agg1p, degp, x, Wl1, Wr1, b1.reshape(1, -1), Wl2, Wr2)
    agg2p = _edge_pass_l2(p2, src, dst, zeros32)
    p3, r3 = _tc2(agg2p, degp, r2, b2.reshape(1, -1), Wl3, Wr3)
    agg3p = _edge_pass_l3(p3, src, dst, zeros16f)
    return _tc3(agg3p, degp, r3, b3.reshape(1, -1),
                fcW1, fcb1.reshape(1, -1), fcW2, fcb2.reshape(1, -1))


# speedup vs baseline: 11.7343x; 1.7219x over previous
import jax
import jax.numpy as jnp
from jax import lax
from jax.experimental import pallas as pl
from jax.experimental.pallas import tpu as pltpu
from jax.experimental.pallas import tpu_sc as plsc

_N = 10000
_NP = 10240
_E = 320000
_NC = 2
_NS = 16
_NW = _NC * _NS
_EPW = _E // _NW
_CH = 80
_RPT = _NP // _NS
_DA = 144


def _make_edge_pass(D, ch, kb):
    ng = _EPW // (kb * ch)
    assert ng * kb * ch == _EPW and ch % 8 == 0 and ch <= 128
    mesh = plsc.VectorSubcoreMesh(
        core_axis_name="c", subcore_axis_name="s",
        num_cores=_NC, num_subcores=_NS)
    scratch = [
        pltpu.VMEM((kb * ch,), jnp.int32),
        pltpu.VMEM((kb, ch), jnp.int32),
        pltpu.VMEM((kb, ch, D), jnp.float32),
        pltpu.VMEM_SHARED((_NP, D), jnp.float32),
        pltpu.SemaphoreType.DMA,
        pltpu.SemaphoreType.DMA,
    ]

    def body(feat, srcs, dst2, zeros, out,
             src_v, dst_v, rows_v, acc_sh, gsem, ssem):
        c = lax.axis_index("c")
        s = lax.axis_index("s")
        wid = c * _NS + s
        r0 = s * _RPT
        pltpu.sync_copy(zeros.at[pl.ds(r0, _RPT)], acc_sh.at[pl.ds(r0, _RPT)])
        plsc.subcore_barrier()
        e0 = wid * _EPW

        def group(g, carry):
            off = e0 + g * kb * ch
            pltpu.sync_copy(srcs.at[pl.ds(off, kb * ch)], src_v)
            pltpu.sync_copy(dst2.at[pl.ds(off // ch, kb)], dst_v)
            gd = [pltpu.async_copy(feat.at[src_v.at[pl.ds(k * ch, ch)]],
                                   rows_v.at[k], gsem) for k in range(kb)]
            for d in gd:
                d.wait()
            sd = [pltpu.async_copy(rows_v.at[k], acc_sh.at[dst_v.at[k]],
                                   ssem, add=True) for k in range(kb)]
            for d in sd:
                d.wait()
            return carry

        lax.fori_loop(0, ng, group, 0)
        plsc.subcore_barrier()
        pltpu.sync_copy(acc_sh.at[pl.ds(r0, _RPT)],
                        out.at[c, pl.ds(r0, _RPT)])

    return pl.kernel(body,
                     out_type=jax.ShapeDtypeStruct((_NC, _NP, D),
                                                   jnp.float32),
                     mesh=mesh, scratch_types=scratch,
                     compiler_params=pltpu.CompilerParams(
                         use_tc_tiling_on_sc=False))


_edge_pass_l1 = _make_edge_pass(_DA, 40, 5)
_edge_pass_l2 = _make_edge_pass(32, 80, 5)
_edge_pass_l3 = _make_edge_pass(16, 80, 5)

_BN = 1000


def _row_spec(w):
    return pl.BlockSpec((_BN, w), lambda i: (i, 0))


def _part_spec(core, w):
    return pl.BlockSpec((1, _BN, w), lambda i, c=core: (c, i, 0))


def _deg_spec(core):
    return pl.BlockSpec((1, _BN, _DA), lambda i, c=core: (c, i, 0))


def _full_spec(shape):
    return pl.BlockSpec(shape, lambda i: tuple(0 for _ in shape))


def _inv_deg(d0, d1):
    deg = d0[0, :, 128:129] + d1[0, :, 128:129]
    return 1.0 / jnp.maximum(deg, 1.0)


def _tc1_body(p0, p1, xr, wl1, wr1, b1, wl2, wr2, p2o, r2o):
    inv = 1.0 / jnp.maximum(p0[0, :, 128:129] + p1[0, :, 128:129], 1.0)
    agg = (p0[0, :, :128] + p1[0, :, :128]) * inv
    h = jnp.dot(agg, wl1[:], preferred_element_type=jnp.float32)
    h = h + jnp.dot(xr[:], wr1[:], preferred_element_type=jnp.float32)
    h = jnp.maximum(h + b1[:], 0.0)
    p2o[:] = jnp.dot(h, wl2[:], preferred_element_type=jnp.float32)
    r2o[:] = jnp.dot(h, wr2[:], preferred_element_type=jnp.float32)


def _tc1(parts, x, Wl1, Wr1, b1, Wl2, Wr2):
    return pl.pallas_call(
        _tc1_body,
        grid=(_N // _BN,),
        in_specs=[_part_spec(0, _DA), _part_spec(1, _DA),
                  _row_spec(128),
                  _full_spec((128, 256)), _full_spec((128, 256)),
                  _full_spec((1, 256)),
                  _full_spec((256, 32)), _full_spec((256, 32))],
        out_specs=[_row_spec(32), _row_spec(32)],
        out_shape=[jax.ShapeDtypeStruct((_N, 32), jnp.float32),
                   jax.ShapeDtypeStruct((_N, 32), jnp.float32)],
    )(parts, parts, x, Wl1, Wr1, b1, Wl2, Wr2)


def _tc2_body(q0, q1, d0, d1, r2, b2, wl3, wr3, p3o, r3o):
    inv = _inv_deg(d0, d1)
    h = jnp.maximum((q0[0] + q1[0]) * inv + b2[:] + r2[:], 0.0)
    p3o[:] = jnp.dot(h, wl3[:], preferred_element_type=jnp.float32)
    r3o[:] = jnp.dot(h, wr3[:], preferred_element_type=jnp.float32)


def _tc2(parts, degp, r2, b2, Wl3, Wr3):
    return pl.pallas_call(
        _tc2_body,
        grid=(_N // _BN,),
        in_specs=[_part_spec(0, 32), _part_spec(1, 32),
                  _deg_spec(0), _deg_spec(1),
                  _row_spec(32), _full_spec((1, 32)),
                  _full_spec((32, 16)), _full_spec((32, 16))],
        out_specs=[_row_spec(16), _row_spec(16)],
        out_shape=[jax.ShapeDtypeStruct((_N, 16), jnp.float32),
                   jax.ShapeDtypeStruct((_N, 16), jnp.float32)],
    )(parts, parts, degp, degp, r2, b2, Wl3, Wr3)


def _tc3_body(t0, t1, d0, d1, r3, b3, w1, bb1, w2, bb2, outo):
    inv = _inv_deg(d0, d1)
    h = jnp.maximum((t0[0] + t1[0]) * inv + b3[:] + r3[:], 0.0)
    h = jnp.maximum(jnp.dot(h, w1[:], preferred_element_type=jnp.float32)
                    + bb1[:], 0.0)
    logits = jnp.dot(h, w2[:], preferred_element_type=jnp.float32) + bb2[:]
    m = jnp.max(logits, axis=1, keepdims=True)
    z = logits - m
    lse = jnp.log(jnp.sum(jnp.exp(z), axis=1, keepdims=True))
    outo[:] = z - lse


def _tc3(parts, degp, r3, b3, fcW1, fcb1, fcW2, fcb2):
    return pl.pallas_call(
        _tc3_body,
        grid=(_N // _BN,),
        in_specs=[_part_spec(0, 16), _part_spec(1, 16),
                  _deg_spec(0), _deg_spec(1),
                  _row_spec(16), _full_spec((1, 16)),
                  _full_spec((16, 8)), _full_spec((1, 8)),
                  _full_spec((8, 2)), _full_spec((1, 2))],
        out_specs=[_row_spec(2)],
        out_shape=[jax.ShapeDtypeStruct((_N, 2), jnp.float32)],
    )(parts, parts, degp, degp, r3, b3, fcW1, fcb1, fcW2, fcb2)[0]


def kernel(x, edge_index, Wl1, Wr1, b1, Wl2, Wr2, b2, Wl3, Wr3, b3,
           fcW1, fcb1, fcW2, fcb2):
    src = edge_index[0].astype(jnp.int32)
    dst = edge_index[1].astype(jnp.int32)
    dst2_40 = dst.reshape(_E // 40, 40)
    dst2_80 = dst.reshape(_E // 80, 80)
    xa = jnp.concatenate(
        [x, jnp.ones((_N, _DA - 128), jnp.float32)], axis=1)
    zerosA = jnp.zeros((_NP, _DA), jnp.float32)
    zeros32 = jnp.zeros((_NP, 32), jnp.float32)
    zeros16f = jnp.zeros((_NP, 16), jnp.float32)

    agg1p = _edge_pass_l1(xa, src, dst2_40, zerosA)
    p2, r2 = _tc1(agg1p, x, Wl1, Wr1, b1.reshape(1, -1), Wl2, Wr2)
    agg2p = _edge_pass_l2(p2, src, dst2_80, zeros32)
    p3, r3 = _tc2(agg2p, agg1p, r2, b2.reshape(1, -1), Wl3, Wr3)
    agg3p = _edge_pass_l3(p3, src, dst2_80, zeros16f)
    return _tc3(agg3p, agg1p, r3, b3.reshape(1, -1),
                fcW1, fcb1.reshape(1, -1), fcW2, fcb2.reshape(1, -1))


# speedup vs baseline: 14.5429x; 1.2394x over previous
import jax
import jax.numpy as jnp
from jax import lax
from jax.experimental import pallas as pl
from jax.experimental.pallas import tpu as pltpu
from jax.experimental.pallas import tpu_sc as plsc

_N = 10000
_NP = 10240
_E = 320000
_NC = 2
_NS = 16
_NW = _NC * _NS
_EPW = _E // _NW
_CH = 80
_RPT = _NP // _NS
_DA = 144


def _make_edge_pass(D, ch, kb):
    kbA = (kb + 1) // 2
    kbB = kb - kbA
    gsz = kb * ch
    ng = _EPW // gsz
    assert ng * gsz == _EPW and ch % 8 == 0 and ch <= 128 and ng >= 2
    mesh = plsc.VectorSubcoreMesh(
        core_axis_name="c", subcore_axis_name="s",
        num_cores=_NC, num_subcores=_NS)
    scratch = [
        pltpu.VMEM((2, gsz), jnp.int32),
        pltpu.VMEM((2, kb, ch), jnp.int32),
        pltpu.VMEM((kbA, ch, D), jnp.float32),
        pltpu.VMEM((kbB, ch, D), jnp.float32),
        pltpu.VMEM_SHARED((_NP, D), jnp.float32),
        pltpu.SemaphoreType.DMA,
        pltpu.SemaphoreType.DMA,
        pltpu.SemaphoreType.DMA,
        pltpu.SemaphoreType.DMA,
        pltpu.SemaphoreType.DMA,
    ]

    def body(feat, srcs, dst2, zeros, out,
             src_v, dst_v, rows_a, rows_b, acc_sh,
             isem, gsemA, gsemB, ssemA, ssemB):
        c = lax.axis_index("c")
        s = lax.axis_index("s")
        wid = c * _NS + s
        r0 = s * _RPT
        pltpu.sync_copy(zeros.at[pl.ds(r0, _RPT)], acc_sh.at[pl.ds(r0, _RPT)])
        plsc.subcore_barrier()
        e0 = wid * _EPW

        def fire_idx(g, b, sync):
            off = e0 + g * gsz
            if sync:
                pltpu.sync_copy(srcs.at[pl.ds(off, gsz)], src_v.at[b])
                pltpu.sync_copy(dst2.at[pl.ds(off // ch, kb)], dst_v.at[b])
            else:
                pltpu.async_copy(srcs.at[pl.ds(off, gsz)], src_v.at[b], isem)
                pltpu.async_copy(dst2.at[pl.ds(off // ch, kb)],
                                 dst_v.at[b], isem)

        def drain_idx(b):
            pltpu.make_async_copy(srcs.at[pl.ds(0, gsz)],
                                  src_v.at[b], isem).wait()
            pltpu.make_async_copy(dst2.at[pl.ds(0, kb)],
                                  dst_v.at[b], isem).wait()

        def fire_gathers(p, k0, rows, kn, sem):
            return [pltpu.async_copy(
                feat.at[src_v.at[p, pl.ds((k0 + k) * ch, ch)]],
                rows.at[k], sem) for k in range(kn)]

        def drain_via(rows, kn, sem):
            for k in range(kn):
                pltpu.make_async_copy(zeros.at[pl.ds(0, ch)],
                                      rows.at[k], sem).wait()

        def fire_scatters(p, k0, rows, kn, sem):
            return [pltpu.async_copy(
                rows.at[k], acc_sh.at[dst_v.at[p, k0 + k]],
                sem, add=True) for k in range(kn)]

        def steady(g, first):
            p = lax.rem(g, 2)
            w = 1 - p
            if not first:
                drain_via(rows_b, kbB, ssemB)
                fire_idx(lax.rem(g + 1, ng), w, sync=False)
            drain_via(rows_a, kbA, gsemA)
            sa = fire_scatters(p, 0, rows_a, kbA, ssemA)
            gb = fire_gathers(p, kbA, rows_b, kbB, gsemB)
            for d in gb:
                d.wait()
            fire_scatters(p, kbA, rows_b, kbB, ssemB)
            for d in sa:
                d.wait()
            drain_idx(w)
            fire_gathers(w, 0, rows_a, kbA, gsemA)

        fire_idx(0, 0, sync=True)
        fire_gathers(0, 0, rows_a, kbA, gsemA)
        fire_idx(1, 1, sync=False)
        steady(0, True)

        def group_body(g, carry):
            steady(g, False)
            return carry

        lax.fori_loop(1, ng, group_body, 0)
        drain_via(rows_b, kbB, ssemB)
        drain_via(rows_a, kbA, gsemA)
        plsc.subcore_barrier()
        pltpu.sync_copy(acc_sh.at[pl.ds(r0, _RPT)],
                        out.at[c, pl.ds(r0, _RPT)])

    return pl.kernel(body,
                     out_type=jax.ShapeDtypeStruct((_NC, _NP, D),
                                                   jnp.float32),
                     mesh=mesh, scratch_types=scratch,
                     compiler_params=pltpu.CompilerParams(
                         use_tc_tiling_on_sc=False))


_edge_pass_l1 = _make_edge_pass(_DA, 40, 5)
_edge_pass_l2 = _make_edge_pass(32, 80, 5)
_edge_pass_l3 = _make_edge_pass(16, 80, 5)

_BN = 1000


def _row_spec(w):
    return pl.BlockSpec((_BN, w), lambda i: (i, 0))


def _part_spec(core, w):
    return pl.BlockSpec((1, _BN, w), lambda i, c=core: (c, i, 0))


def _deg_spec(core):
    return pl.BlockSpec((1, _BN, _DA), lambda i, c=core: (c, i, 0))


def _full_spec(shape):
    return pl.BlockSpec(shape, lambda i: tuple(0 for _ in shape))


def _inv_deg(d0, d1):
    deg = d0[0, :, 128:129] + d1[0, :, 128:129]
    return 1.0 / jnp.maximum(deg, 1.0)


def _tc1_body(p0, p1, xr, wl1, wr1, b1, wl2, wr2, p2o, r2o):
    inv = 1.0 / jnp.maximum(p0[0, :, 128:129] + p1[0, :, 128:129], 1.0)
    agg = (p0[0, :, :128] + p1[0, :, :128]) * inv
    h = jnp.dot(agg, wl1[:], preferred_element_type=jnp.float32)
    h = h + jnp.dot(xr[:], wr1[:], preferred_element_type=jnp.float32)
    h = jnp.maximum(h + b1[:], 0.0)
    p2o[:] = jnp.dot(h, wl2[:], preferred_element_type=jnp.float32)
    r2o[:] = jnp.dot(h, wr2[:], preferred_element_type=jnp.float32)


def _tc1(parts, x, Wl1, Wr1, b1, Wl2, Wr2):
    return pl.pallas_call(
        _tc1_body,
        grid=(_N // _BN,),
        in_specs=[_part_spec(0, _DA), _part_spec(1, _DA),
                  _row_spec(128),
                  _full_spec((128, 256)), _full_spec((128, 256)),
                  _full_spec((1, 256)),
                  _full_spec((256, 32)), _full_spec((256, 32))],
        out_specs=[_row_spec(32), _row_spec(32)],
        out_shape=[jax.ShapeDtypeStruct((_N, 32), jnp.float32),
                   jax.ShapeDtypeStruct((_N, 32), jnp.float32)],
    )(parts, parts, x, Wl1, Wr1, b1, Wl2, Wr2)


def _tc2_body(q0, q1, d0, d1, r2, b2, wl3, wr3, p3o, r3o):
    inv = _inv_deg(d0, d1)
    h = jnp.maximum((q0[0] + q1[0]) * inv + b2[:] + r2[:], 0.0)
    p3o[:] = jnp.dot(h, wl3[:], preferred_element_type=jnp.float32)
    r3o[:] = jnp.dot(h, wr3[:], preferred_element_type=jnp.float32)


def _tc2(parts, degp, r2, b2, Wl3, Wr3):
    return pl.pallas_call(
        _tc2_body,
        grid=(_N // _BN,),
        in_specs=[_part_spec(0, 32), _part_spec(1, 32),
                  _deg_spec(0), _deg_spec(1),
                  _row_spec(32), _full_spec((1, 32)),
                  _full_spec((32, 16)), _full_spec((32, 16))],
        out_specs=[_row_spec(16), _row_spec(16)],
        out_shape=[jax.ShapeDtypeStruct((_N, 16), jnp.float32),
                   jax.ShapeDtypeStruct((_N, 16), jnp.float32)],
    )(parts, parts, degp, degp, r2, b2, Wl3, Wr3)


def _tc3_body(t0, t1, d0, d1, r3, b3, w1, bb1, w2, bb2, outo):
    inv = _inv_deg(d0, d1)
    h = jnp.maximum((t0[0] + t1[0]) * inv + b3[:] + r3[:], 0.0)
    h = jnp.maximum(jnp.dot(h, w1[:], preferred_element_type=jnp.float32)
                    + bb1[:], 0.0)
    logits = jnp.dot(h, w2[:], preferred_element_type=jnp.float32) + bb2[:]
    m = jnp.max(logits, axis=1, keepdims=True)
    z = logits - m
    lse = jnp.log(jnp.sum(jnp.exp(z), axis=1, keepdims=True))
    outo[:] = z - lse


def _tc3(parts, degp, r3, b3, fcW1, fcb1, fcW2, fcb2):
    return pl.pallas_call(
        _tc3_body,
        grid=(_N // _BN,),
        in_specs=[_part_spec(0, 16), _part_spec(1, 16),
                  _deg_spec(0), _deg_spec(1),
                  _row_spec(16), _full_spec((1, 16)),
                  _full_spec((16, 8)), _full_spec((1, 8)),
                  _full_spec((8, 2)), _full_spec((1, 2))],
        out_specs=[_row_spec(2)],
        out_shape=[jax.ShapeDtypeStruct((_N, 2), jnp.float32)],
    )(parts, parts, degp, degp, r3, b3, fcW1, fcb1, fcW2, fcb2)[0]


def kernel(x, edge_index, Wl1, Wr1, b1, Wl2, Wr2, b2, Wl3, Wr3, b3,
           fcW1, fcb1, fcW2, fcb2):
    src = edge_index[0].astype(jnp.int32)
    dst = edge_index[1].astype(jnp.int32)
    dst2_40 = dst.reshape(_E // 40, 40)
    dst2_80 = dst.reshape(_E // 80, 80)
    xa = jnp.concatenate(
        [x, jnp.ones((_N, _DA - 128), jnp.float32)], axis=1)
    zerosA = jnp.zeros((_NP, _DA), jnp.float32)
    zeros32 = jnp.zeros((_NP, 32), jnp.float32)
    zeros16f = jnp.zeros((_NP, 16), jnp.float32)

    agg1p = _edge_pass_l1(xa, src, dst2_40, zerosA)
    p2, r2 = _tc1(agg1p, x, Wl1, Wr1, b1.reshape(1, -1), Wl2, Wr2)
    agg2p = _edge_pass_l2(p2, src, dst2_80, zeros32)
    p3, r3 = _tc2(agg2p, agg1p, r2, b2.reshape(1, -1), Wl3, Wr3)
    agg3p = _edge_pass_l3(p3, src, dst2_80, zeros16f)
    return _tc3(agg3p, agg1p, r3, b3.reshape(1, -1),
                fcW1, fcb1.reshape(1, -1), fcW2, fcb2.reshape(1, -1))


# speedup vs baseline: 16.0198x; 1.1016x over previous
import jax
import jax.numpy as jnp
from jax import lax
from jax.experimental import pallas as pl
from jax.experimental.pallas import tpu as pltpu
from jax.experimental.pallas import tpu_sc as plsc

_N = 10000
_NP = 10240
_E = 320000
_NC = 2
_NS = 16
_NW = _NC * _NS
_EPW = _E // _NW
_RPT = _NP // _NS
_DEGW = 16


def _make_edge_pass(D, ch, kb, with_deg):
    kbA = (kb + 1) // 2
    kbB = kb - kbA
    gsz = kb * ch
    ng = _EPW // gsz
    assert ng * gsz == _EPW and ch % 8 == 0 and ch <= 128 and ng >= 2
    mesh = plsc.VectorSubcoreMesh(
        core_axis_name="c", subcore_axis_name="s",
        num_cores=_NC, num_subcores=_NS)
    scratch = [
        pltpu.VMEM((2, gsz), jnp.int32),
        pltpu.VMEM((2, kb, ch), jnp.int32),
        pltpu.VMEM((kbA, ch, D), jnp.float32),
        pltpu.VMEM((kbB, ch, D), jnp.float32),
        pltpu.VMEM_SHARED((_NP, D), jnp.float32),
        pltpu.SemaphoreType.DMA,
        pltpu.SemaphoreType.DMA,
        pltpu.SemaphoreType.DMA,
        pltpu.SemaphoreType.DMA,
        pltpu.SemaphoreType.DMA,
    ]
    out_type = [jax.ShapeDtypeStruct((_NC, _NP, D), jnp.float32)]
    if with_deg:
        scratch += [
            pltpu.VMEM((ch, _DEGW), jnp.float32),
            pltpu.VMEM_SHARED((_NP, _DEGW), jnp.float32),
        ]
        out_type.append(jax.ShapeDtypeStruct((_NC, _NP, _DEGW), jnp.float32))

    def body(*refs):
        if with_deg:
            (feat, srcs, dst2, zeros, zeros16, out, degout,
             src_v, dst_v, rows_a, rows_b, acc_sh,
             isem, gsemA, gsemB, ssemA, ssemB, ones_v, deg_sh) = refs
        else:
            (feat, srcs, dst2, zeros, out,
             src_v, dst_v, rows_a, rows_b, acc_sh,
             isem, gsemA, gsemB, ssemA, ssemB) = refs
        c = lax.axis_index("c")
        s = lax.axis_index("s")
        wid = c * _NS + s
        r0 = s * _RPT
        pltpu.sync_copy(zeros.at[pl.ds(r0, _RPT)], acc_sh.at[pl.ds(r0, _RPT)])
        if with_deg:
            pltpu.sync_copy(zeros16.at[pl.ds(r0, _RPT)],
                            deg_sh.at[pl.ds(r0, _RPT)])
            pltpu.sync_copy(zeros16.at[pl.ds(0, ch)], ones_v)

            def fill_ones(i, carry):
                ones_v[i] = jnp.ones((_DEGW,), jnp.float32)
                return carry

            lax.fori_loop(0, ch, fill_ones, 0)
        plsc.subcore_barrier()
        e0 = wid * _EPW

        def fire_idx(g, b, sync):
            off = e0 + g * gsz
            if sync:
                pltpu.sync_copy(srcs.at[pl.ds(off, gsz)], src_v.at[b])
                pltpu.sync_copy(dst2.at[pl.ds(off // ch, kb)], dst_v.at[b])
            else:
                pltpu.async_copy(srcs.at[pl.ds(off, gsz)], src_v.at[b], isem)
                pltpu.async_copy(dst2.at[pl.ds(off // ch, kb)],
                                 dst_v.at[b], isem)

        def drain_idx(b):
            pltpu.make_async_copy(srcs.at[pl.ds(0, gsz)],
                                  src_v.at[b], isem).wait()
            pltpu.make_async_copy(dst2.at[pl.ds(0, kb)],
                                  dst_v.at[b], isem).wait()

        def fire_gathers(p, k0, rows, kn, sem):
            return [pltpu.async_copy(
                feat.at[src_v.at[p, pl.ds((k0 + k) * ch, ch)]],
                rows.at[k], sem) for k in range(kn)]

        def drain_rows(rows, kn, sem):
            for k in range(kn):
                pltpu.make_async_copy(zeros.at[pl.ds(0, ch)],
                                      rows.at[k], sem).wait()

        def fire_scatters(p, k0, rows, kn, sem):
            d = [pltpu.async_copy(
                rows.at[k], acc_sh.at[dst_v.at[p, k0 + k]],
                sem, add=True) for k in range(kn)]
            if with_deg:
                d += [pltpu.async_copy(
                    ones_v, deg_sh.at[dst_v.at[p, k0 + k]],
                    sem, add=True) for k in range(kn)]
            return d

        def drain_deg(kn, sem):
            if with_deg:
                for k in range(kn):
                    pltpu.make_async_copy(zeros16.at[pl.ds(0, ch)],
                                          ones_v, sem).wait()

        def steady(g, first):
            p = lax.rem(g, 2)
            w = 1 - p
            if not first:
                drain_rows(rows_b, kbB, ssemB)
                drain_deg(kbB, ssemB)
                fire_idx(lax.rem(g + 1, ng), w, sync=False)
            drain_rows(rows_a, kbA, gsemA)
            sa = fire_scatters(p, 0, rows_a, kbA, ssemA)
            gb = fire_gathers(p, kbA, rows_b, kbB, gsemB)
            for d in gb:
                d.wait()
            fire_scatters(p, kbA, rows_b, kbB, ssemB)
            for d in sa:
                d.wait()
            drain_idx(w)
            fire_gathers(w, 0, rows_a, kbA, gsemA)

        fire_idx(0, 0, sync=True)
        fire_gathers(0, 0, rows_a, kbA, gsemA)
        fire_idx(1, 1, sync=False)
        steady(0, True)

        def group_body(g, carry):
            steady(g, False)
            return carry

        lax.fori_loop(1, ng, group_body, 0)
        drain_rows(rows_b, kbB, ssemB)
        drain_deg(kbB, ssemB)
        drain_rows(rows_a, kbA, gsemA)
        plsc.subcore_barrier()
        pltpu.sync_copy(acc_sh.at[pl.ds(r0, _RPT)],
                        out.at[c, pl.ds(r0, _RPT)])
        if with_deg:
            pltpu.sync_copy(deg_sh.at[pl.ds(r0, _RPT)],
                            degout.at[c, pl.ds(r0, _RPT)])

    return pl.kernel(body,
                     out_type=tuple(out_type) if with_deg else out_type[0],
                     mesh=mesh, scratch_types=scratch,
                     compiler_params=pltpu.CompilerParams(
                         use_tc_tiling_on_sc=False))


_edge_pass_l1 = _make_edge_pass(128, 40, 5, True)
_edge_pass_l2 = _make_edge_pass(32, 80, 5, False)
_edge_pass_l3 = _make_edge_pass(16, 80, 5, False)

_BN = 1024


def _row_spec(w):
    return pl.BlockSpec((_BN, w), lambda i: (i, 0))


def _part_spec(core, rows, w=128):
    return pl.BlockSpec((1, rows, w), lambda i, c=core: (c, i, 0))


def _packed_spec(rows):
    return pl.BlockSpec((rows, 128), lambda i: (i, 0))


def _full_spec(shape):
    return pl.BlockSpec(shape, lambda i: tuple(0 for _ in shape))


def _tc1_body(f0, f1, d0, d1, xr, wl1, wr1, b1, wl2, wr2,
              p2o, r2o, invo):
    degs = d0[0] + d1[0]
    inv = 1.0 / jnp.maximum(degs[:, 0:1], 1.0)
    agg = (f0[0] + f1[0]) * inv
    h = jnp.dot(agg, wl1[:], preferred_element_type=jnp.float32)
    h = h + jnp.dot(xr[:], wr1[:], preferred_element_type=jnp.float32)
    h = jnp.maximum(h + b1[:], 0.0)
    p2o[:] = jnp.dot(h, wl2[:], preferred_element_type=jnp.float32)
    r2o[:] = jnp.dot(h, wr2[:], preferred_element_type=jnp.float32)
    invo[:] = jnp.broadcast_to(inv, (_BN, _DEGW))


def _tc1(featp, degp, x, Wl1, Wr1, b1, Wl2, Wr2):
    return pl.pallas_call(
        _tc1_body,
        grid=(_NP // _BN,),
        in_specs=[_part_spec(0, _BN), _part_spec(1, _BN),
                  _part_spec(0, _BN, _DEGW), _part_spec(1, _BN, _DEGW),
                  _row_spec(128),
                  _full_spec((128, 256)), _full_spec((128, 256)),
                  _full_spec((1, 256)),
                  _full_spec((256, 32)), _full_spec((256, 32))],
        out_specs=[_row_spec(32), _row_spec(32), _row_spec(_DEGW)],
        out_shape=[jax.ShapeDtypeStruct((_NP, 32), jnp.float32),
                   jax.ShapeDtypeStruct((_NP, 32), jnp.float32),
                   jax.ShapeDtypeStruct((_NP, _DEGW), jnp.float32)],
    )(featp, featp, degp, degp, x, Wl1, Wr1, b1, Wl2, Wr2)


def _tc2_body(q0, q1, iv, r2, b2, wl3, wr3, p3o, r3o):
    inv = iv[:, 0:1]
    h = jnp.maximum((q0[0] + q1[0]) * inv + b2[:] + r2[:], 0.0)
    p3o[:] = jnp.dot(h, wl3[:], preferred_element_type=jnp.float32)
    r3o[:] = jnp.dot(h, wr3[:], preferred_element_type=jnp.float32)


def _tc2(aggp, invb, r2, b2, Wl3, Wr3):
    return pl.pallas_call(
        _tc2_body,
        grid=(_NP // _BN,),
        in_specs=[_part_spec(0, _BN, 32), _part_spec(1, _BN, 32),
                  _row_spec(_DEGW), _row_spec(32),
                  _full_spec((1, 32)),
                  _full_spec((32, 16)), _full_spec((32, 16))],
        out_specs=[_row_spec(16), _row_spec(16)],
        out_shape=[jax.ShapeDtypeStruct((_NP, 16), jnp.float32),
                   jax.ShapeDtypeStruct((_NP, 16), jnp.float32)],
    )(aggp, aggp, invb, r2, b2, Wl3, Wr3)


def _tc3_body(t0, t1, iv, r3, b3, w1, bb1, w2, bb2, outo):
    inv = iv[:, 0:1]
    h = jnp.maximum((t0[0] + t1[0]) * inv + b3[:] + r3[:], 0.0)
    h = jnp.maximum(jnp.dot(h, w1[:], preferred_element_type=jnp.float32)
                    + bb1[:], 0.0)
    logits = jnp.dot(h, w2[:], preferred_element_type=jnp.float32) + bb2[:]
    m = jnp.max(logits, axis=1, keepdims=True)
    z = logits - m
    lse = jnp.log(jnp.sum(jnp.exp(z), axis=1, keepdims=True))
    outo[:] = z - lse


def _tc3(aggp, invb, r3, b3, fcW1, fcb1, fcW2, fcb2):
    return pl.pallas_call(
        _tc3_body,
        grid=(_NP // _BN,),
        in_specs=[_part_spec(0, _BN, 16), _part_spec(1, _BN, 16),
                  _row_spec(_DEGW), _row_spec(16),
                  _full_spec((1, 16)),
                  _full_spec((16, 8)), _full_spec((1, 8)),
                  _full_spec((8, 2)), _full_spec((1, 2))],
        out_specs=[_row_spec(2)],
        out_shape=[jax.ShapeDtypeStruct((_N, 2), jnp.float32)],
    )(aggp, aggp, invb, r3, b3, fcW1, fcb1, fcW2, fcb2)[0]


def kernel(x, edge_index, Wl1, Wr1, b1, Wl2, Wr2, b2, Wl3, Wr3, b3,
           fcW1, fcb1, fcW2, fcb2):
    src = edge_index[0].astype(jnp.int32)
    dst = edge_index[1].astype(jnp.int32)
    dst2_40 = dst.reshape(_E // 40, 40)
    dst2_80 = dst.reshape(_E // 80, 80)
    zeros128 = jnp.zeros((_NP, 128), jnp.float32)
    zeros32 = jnp.zeros((_NP, 32), jnp.float32)
    zeros16f = jnp.zeros((_NP, 16), jnp.float32)

    featp, degp = _edge_pass_l1(x, src, dst2_40, zeros128, zeros16f)
    xp = jnp.pad(x, ((0, _NP - _N), (0, 0)))
    p2, r2, invb = _tc1(featp, degp, xp, Wl1, Wr1,
                        b1.reshape(1, -1), Wl2, Wr2)
    agg2p = _edge_pass_l2(p2, src, dst2_80, zeros32)
    p3, r3 = _tc2(agg2p, invb, r2, b2.reshape(1, -1), Wl3, Wr3)
    agg3p = _edge_pass_l3(p3, src, dst2_80, zeros16f)
    return _tc3(agg3p, invb, r3, b3.reshape(1, -1),
                fcW1, fcb1.reshape(1, -1), fcW2, fcb2.reshape(1, -1))


# speedup vs baseline: 16.6721x; 1.0407x over previous
import jax
import jax.numpy as jnp
from jax import lax
from jax.experimental import pallas as pl
from jax.experimental.pallas import tpu as pltpu
from jax.experimental.pallas import tpu_sc as plsc

_N = 10000
_NP = 10240
_E = 320000
_NC = 2
_NS = 16
_NW = _NC * _NS
_EPW = _E // _NW
_RPT = _NP // _NS
_DEGW = 16


def _make_edge_pass(D, ch, kb, with_deg):
    kbA = (kb + 1) // 2
    kbB = kb - kbA
    gsz = kb * ch
    ng = _EPW // gsz
    assert ng * gsz == _EPW and ch % 8 == 0 and ch <= 128 and ng >= 2
    mesh = plsc.VectorSubcoreMesh(
        core_axis_name="c", subcore_axis_name="s",
        num_cores=_NC, num_subcores=_NS)
    scratch = [
        pltpu.VMEM((2, gsz), jnp.int32),
        pltpu.VMEM((2, kb, ch), jnp.int32),
        pltpu.VMEM((kbA, ch, D), jnp.float32),
        pltpu.VMEM((kbB, ch, D), jnp.float32),
        pltpu.VMEM_SHARED((_NP, D), jnp.float32),
        pltpu.SemaphoreType.DMA,
        pltpu.SemaphoreType.DMA,
        pltpu.SemaphoreType.DMA,
        pltpu.SemaphoreType.DMA,
        pltpu.SemaphoreType.DMA,
    ]
    out_type = [jax.ShapeDtypeStruct((_NC, _NP, D), jnp.float32)]
    if with_deg:
        scratch += [
            pltpu.VMEM((ch, _DEGW), jnp.float32),
            pltpu.VMEM_SHARED((_NP, _DEGW), jnp.float32),
        ]
        out_type.append(jax.ShapeDtypeStruct((_NC, _NP, _DEGW), jnp.float32))

    def body(*refs):
        if with_deg:
            (feat, srcs, dst2, zeros, zeros16, out, degout,
             src_v, dst_v, rows_a, rows_b, acc_sh,
             isem, gsemA, gsemB, ssemA, ssemB, ones_v, deg_sh) = refs
        else:
            (feat, srcs, dst2, zeros, out,
             src_v, dst_v, rows_a, rows_b, acc_sh,
             isem, gsemA, gsemB, ssemA, ssemB) = refs
        c = lax.axis_index("c")
        s = lax.axis_index("s")
        wid = c * _NS + s
        r0 = s * _RPT
        pltpu.sync_copy(zeros.at[pl.ds(r0, _RPT)], acc_sh.at[pl.ds(r0, _RPT)])
        if with_deg:
            pltpu.sync_copy(zeros16.at[pl.ds(r0, _RPT)],
                            deg_sh.at[pl.ds(r0, _RPT)])
            pltpu.sync_copy(zeros16.at[pl.ds(0, ch)], ones_v)

            def fill_ones(i, carry):
                ones_v[i] = jnp.ones((_DEGW,), jnp.float32)
                return carry

            lax.fori_loop(0, ch, fill_ones, 0)
        plsc.subcore_barrier()
        e0 = wid * _EPW

        def fire_idx(g, b, sync):
            off = e0 + g * gsz
            if sync:
                pltpu.sync_copy(srcs.at[pl.ds(off, gsz)], src_v.at[b])
                pltpu.sync_copy(dst2.at[pl.ds(off // ch, kb)], dst_v.at[b])
            else:
                pltpu.async_copy(srcs.at[pl.ds(off, gsz)], src_v.at[b], isem)
                pltpu.async_copy(dst2.at[pl.ds(off // ch, kb)],
                                 dst_v.at[b], isem)

        def drain_idx(b):
            pltpu.make_async_copy(srcs.at[pl.ds(0, gsz)],
                                  src_v.at[b], isem).wait()
            pltpu.make_async_copy(dst2.at[pl.ds(0, kb)],
                                  dst_v.at[b], isem).wait()

        def fire_gathers(p, k0, rows, kn, sem):
            return [pltpu.async_copy(
                feat.at[src_v.at[p, pl.ds((k0 + k) * ch, ch)]],
                rows.at[k], sem) for k in range(kn)]

        def drain_rows(rows, kn, sem):
            for k in range(kn):
                pltpu.make_async_copy(zeros.at[pl.ds(0, ch)],
                                      rows.at[k], sem).wait()

        def fire_scatters(p, k0, rows, kn, sem):
            d = [pltpu.async_copy(
                rows.at[k], acc_sh.at[dst_v.at[p, k0 + k]],
                sem, add=True) for k in range(kn)]
            if with_deg:
                d += [pltpu.async_copy(
                    ones_v, deg_sh.at[dst_v.at[p, k0 + k]],
                    sem, add=True) for k in range(kn)]
            return d

        def drain_deg(kn, sem):
            if with_deg:
                for k in range(kn):
                    pltpu.make_async_copy(zeros16.at[pl.ds(0, ch)],
                                          ones_v, sem).wait()

        def steady(g, first):
            p = lax.rem(g, 2)
            w = 1 - p
            if not first:
                drain_rows(rows_b, kbB, ssemB)
                drain_deg(kbB, ssemB)
                fire_idx(lax.rem(g + 1, ng), w, sync=False)
            drain_rows(rows_a, kbA, gsemA)
            sa = fire_scatters(p, 0, rows_a, kbA, ssemA)
            gb = fire_gathers(p, kbA, rows_b, kbB, gsemB)
            for d in gb:
                d.wait()
            fire_scatters(p, kbA, rows_b, kbB, ssemB)
            for d in sa:
                d.wait()
            drain_idx(w)
            fire_gathers(w, 0, rows_a, kbA, gsemA)

        fire_idx(0, 0, sync=True)
        fire_gathers(0, 0, rows_a, kbA, gsemA)
        fire_idx(1, 1, sync=False)
        steady(0, True)

        def group_body(g, carry):
            steady(g, False)
            return carry

        lax.fori_loop(1, ng, group_body, 0)
        drain_rows(rows_b, kbB, ssemB)
        drain_deg(kbB, ssemB)
        drain_rows(rows_a, kbA, gsemA)
        plsc.subcore_barrier()
        pltpu.sync_copy(acc_sh.at[pl.ds(r0, _RPT)],
                        out.at[c, pl.ds(r0, _RPT)])
        if with_deg:
            pltpu.sync_copy(deg_sh.at[pl.ds(r0, _RPT)],
                            degout.at[c, pl.ds(r0, _RPT)])

    return pl.kernel(body,
                     out_type=tuple(out_type) if with_deg else out_type[0],
                     mesh=mesh, scratch_types=scratch,
                     compiler_params=pltpu.CompilerParams(
                         use_tc_tiling_on_sc=False))


_edge_pass_l1 = _make_edge_pass(128, 40, 5, True)
_edge_pass_l2 = _make_edge_pass(32, 80, 5, False)
_edge_pass_l3 = _make_edge_pass(16, 80, 5, False)

_BN = 1024


def _row_spec(w):
    return pl.BlockSpec((_BN, w), lambda i: (i, 0))


def _part_spec(core, rows, w=128):
    return pl.BlockSpec((1, rows, w), lambda i, c=core: (c, i, 0))


def _packed_spec(rows):
    return pl.BlockSpec((rows, 128), lambda i: (i, 0))


def _full_spec(shape):
    return pl.BlockSpec(shape, lambda i: tuple(0 for _ in shape))


def _pack_rows(y, w):
    m = 128 // w
    rows = _BN * w // 128
    ri = lax.broadcasted_iota(jnp.int32, (rows, _BN), 0)
    ni = lax.broadcasted_iota(jnp.int32, (rows, _BN), 1)
    parts = []
    for a in range(m):
        S = (ni == m * ri + a).astype(jnp.float32)
        parts.append(jnp.dot(S, y, preferred_element_type=jnp.float32))
    return jnp.concatenate(parts, axis=1)


def _unpack_rows(pk, w):
    m = 128 // w
    rows = _BN * w // 128
    ni = lax.broadcasted_iota(jnp.int32, (_BN, rows), 0)
    ri = lax.broadcasted_iota(jnp.int32, (_BN, rows), 1)
    A = (ni // m == ri).astype(jnp.float32)
    Y = jnp.dot(A, pk, preferred_element_type=jnp.float32)
    col = lax.broadcasted_iota(jnp.int32, (_BN, 1), 0) % m
    out = jnp.zeros((_BN, w), jnp.float32)
    for a in range(m):
        out = out + Y[:, a * w:(a + 1) * w] * (col == a).astype(jnp.float32)
    return out


def _tc1_body(f0, f1, d0, d1, xr, wl1, wr1, b1, wl2, wr2,
              p2o, r2o, invo):
    degs = _unpack_rows(d0[0] + d1[0], _DEGW)
    inv = 1.0 / jnp.maximum(degs[:, 0:1], 1.0)
    agg = (f0[0] + f1[0]) * inv
    h = jnp.dot(agg, wl1[:], preferred_element_type=jnp.float32)
    h = h + jnp.dot(xr[:], wr1[:], preferred_element_type=jnp.float32)
    h = jnp.maximum(h + b1[:], 0.0)
    p2o[:] = _pack_rows(
        jnp.dot(h, wl2[:], preferred_element_type=jnp.float32), 32)
    r2o[:] = jnp.dot(h, wr2[:], preferred_element_type=jnp.float32)
    invo[:] = jnp.broadcast_to(inv, (_BN, _DEGW))


def _tc1(featp, degpk, x, Wl1, Wr1, b1, Wl2, Wr2):
    n32 = _BN * 32 // 128
    n16 = _BN * 16 // 128
    return pl.pallas_call(
        _tc1_body,
        grid=(_NP // _BN,),
        in_specs=[_part_spec(0, _BN), _part_spec(1, _BN),
                  _part_spec(0, n16), _part_spec(1, n16),
                  _row_spec(128),
                  _full_spec((128, 256)), _full_spec((128, 256)),
                  _full_spec((1, 256)),
                  _full_spec((256, 32)), _full_spec((256, 32))],
        out_specs=[_packed_spec(n32), _row_spec(32), _row_spec(_DEGW)],
        out_shape=[jax.ShapeDtypeStruct((_NP * 32 // 128, 128), jnp.float32),
                   jax.ShapeDtypeStruct((_NP, 32), jnp.float32),
                   jax.ShapeDtypeStruct((_NP, _DEGW), jnp.float32)],
    )(featp, featp, degpk, degpk, x, Wl1, Wr1, b1, Wl2, Wr2)


def _tc2_body(q0, q1, iv, r2, b2, wl3, wr3, p3o, r3o):
    inv = iv[:, 0:1]
    q = _unpack_rows(q0[0] + q1[0], 32)
    h = jnp.maximum(q * inv + b2[:] + r2[:], 0.0)
    p3o[:] = _pack_rows(
        jnp.dot(h, wl3[:], preferred_element_type=jnp.float32), 16)
    r3o[:] = jnp.dot(h, wr3[:], preferred_element_type=jnp.float32)


def _tc2(aggpk, invb, r2, b2, Wl3, Wr3):
    n32 = _BN * 32 // 128
    n16 = _BN * 16 // 128
    return pl.pallas_call(
        _tc2_body,
        grid=(_NP // _BN,),
        in_specs=[_part_spec(0, n32), _part_spec(1, n32),
                  _row_spec(_DEGW), _row_spec(32),
                  _full_spec((1, 32)),
                  _full_spec((32, 16)), _full_spec((32, 16))],
        out_specs=[_packed_spec(n16), _row_spec(16)],
        out_shape=[jax.ShapeDtypeStruct((_NP * 16 // 128, 128), jnp.float32),
                   jax.ShapeDtypeStruct((_NP, 16), jnp.float32)],
    )(aggpk, aggpk, invb, r2, b2, Wl3, Wr3)


def _tc3_body(t0, t1, iv, r3, b3, w1, bb1, w2, bb2, outo):
    inv = iv[:, 0:1]
    t = _unpack_rows(t0[0] + t1[0], 16)
    h = jnp.maximum(t * inv + b3[:] + r3[:], 0.0)
    h = jnp.maximum(jnp.dot(h, w1[:], preferred_element_type=jnp.float32)
                    + bb1[:], 0.0)
    logits = jnp.dot(h, w2[:], preferred_element_type=jnp.float32) + bb2[:]
    m = jnp.max(logits, axis=1, keepdims=True)
    z = logits - m
    lse = jnp.log(jnp.sum(jnp.exp(z), axis=1, keepdims=True))
    outo[:] = z - lse


def _tc3(aggpk, invb, r3, b3, fcW1, fcb1, fcW2, fcb2):
    n16 = _BN * 16 // 128
    return pl.pallas_call(
        _tc3_body,
        grid=(_NP // _BN,),
        in_specs=[_part_spec(0, n16), _part_spec(1, n16),
                  _row_spec(_DEGW), _row_spec(16),
                  _full_spec((1, 16)),
                  _full_spec((16, 8)), _full_spec((1, 8)),
                  _full_spec((8, 2)), _full_spec((1, 2))],
        out_specs=[_row_spec(2)],
        out_shape=[jax.ShapeDtypeStruct((_N, 2), jnp.float32)],
    )(aggpk, aggpk, invb, r3, b3, fcW1, fcb1, fcW2, fcb2)[0]


def kernel(x, edge_index, Wl1, Wr1, b1, Wl2, Wr2, b2, Wl3, Wr3, b3,
           fcW1, fcb1, fcW2, fcb2):
    src = edge_index[0].astype(jnp.int32)
    dst = edge_index[1].astype(jnp.int32)
    dst2_40 = dst.reshape(_E // 40, 40)
    dst2_80 = dst.reshape(_E // 80, 80)
    zeros128 = jnp.zeros((_NP, 128), jnp.float32)
    zeros32 = jnp.zeros((_NP, 32), jnp.float32)
    zeros16f = jnp.zeros((_NP, 16), jnp.float32)

    featp, degp = _edge_pass_l1(x, src, dst2_40, zeros128, zeros16f)
    degpk = degp.reshape(_NC, _NP * _DEGW // 128, 128)
    p2pk, r2, invb = _tc1(featp, degpk, x, Wl1, Wr1,
                          b1.reshape(1, -1), Wl2, Wr2)
    agg2p = _edge_pass_l2(p2pk.reshape(_NP, 32), src, dst2_80, zeros32)
    p3pk, r3 = _tc2(agg2p.reshape(_NC, _NP * 32 // 128, 128), invb, r2,
                    b2.reshape(1, -1), Wl3, Wr3)
    agg3p = _edge_pass_l3(p3pk.reshape(_NP, 16), src, dst2_80, zeros16f)
    return _tc3(agg3p.reshape(_NC, _NP * 16 // 128, 128), invb, r3,
                b3.reshape(1, -1),
                fcW1, fcb1.reshape(1, -1), fcW2, fcb2.reshape(1, -1))


# speedup vs baseline: 18.8199x; 1.1288x over previous
import jax
import jax.numpy as jnp
from jax import lax
from jax.experimental import pallas as pl
from jax.experimental.pallas import tpu as pltpu
from jax.experimental.pallas import tpu_sc as plsc

_N = 10000
_NP = 10240
_E = 320000
_NC = 2
_NS = 16
_NW = _NC * _NS
_EPW = _E // _NW
_RPT = _NP // _NS
_DEGW = 16


def _make_edge_pass(D, ch, kb, with_deg):
    kbA = (kb + 1) // 2
    kbB = kb - kbA
    gsz = kb * ch
    ng = _EPW // gsz
    assert ng * gsz == _EPW and ch % 8 == 0 and ch <= 128 and ng >= 2
    mesh = plsc.VectorSubcoreMesh(
        core_axis_name="c", subcore_axis_name="s",
        num_cores=_NC, num_subcores=_NS)
    scratch = [
        pltpu.VMEM((2, gsz), jnp.int32),
        pltpu.VMEM((2, kb, ch), jnp.int32),
        pltpu.VMEM((kbA, ch, D), jnp.float32),
        pltpu.VMEM((kbB, ch, D), jnp.float32),
        pltpu.VMEM_SHARED((_NP, D), jnp.float32),
        pltpu.SemaphoreType.DMA,
        pltpu.SemaphoreType.DMA,
        pltpu.SemaphoreType.DMA,
        pltpu.SemaphoreType.DMA,
        pltpu.SemaphoreType.DMA,
    ]
    out_type = [jax.ShapeDtypeStruct((_NC, _NP, D), jnp.float32)]
    if with_deg:
        scratch += [
            pltpu.VMEM((ch, _DEGW), jnp.float32),
            pltpu.VMEM_SHARED((_NP, _DEGW), jnp.float32),
        ]
        out_type.append(jax.ShapeDtypeStruct((_NC, _NP, _DEGW), jnp.float32))

    def body(*refs):
        if with_deg:
            (feat, srcs, dst2, zeros, zeros16, out, degout,
             src_v, dst_v, rows_a, rows_b, acc_sh,
             isem, gsemA, gsemB, ssemA, ssemB, ones_v, deg_sh) = refs
        else:
            (feat, srcs, dst2, zeros, out,
             src_v, dst_v, rows_a, rows_b, acc_sh,
             isem, gsemA, gsemB, ssemA, ssemB) = refs
        c = lax.axis_index("c")
        s = lax.axis_index("s")
        wid = c * _NS + s
        r0 = s * _RPT
        pltpu.sync_copy(zeros.at[pl.ds(r0, _RPT)], acc_sh.at[pl.ds(r0, _RPT)])
        if with_deg:
            pltpu.sync_copy(zeros16.at[pl.ds(r0, _RPT)],
                            deg_sh.at[pl.ds(r0, _RPT)])
            pltpu.sync_copy(zeros16.at[pl.ds(0, ch)], ones_v)

            def fill_ones(i, carry):
                ones_v[i] = jnp.ones((_DEGW,), jnp.float32)
                return carry

            lax.fori_loop(0, ch, fill_ones, 0)
        plsc.subcore_barrier()
        e0 = wid * _EPW

        def fire_idx(g, b, sync):
            off = e0 + g * gsz
            if sync:
                pltpu.sync_copy(srcs.at[pl.ds(off, gsz)], src_v.at[b])
                pltpu.sync_copy(dst2.at[pl.ds(off // ch, kb)], dst_v.at[b])
            else:
                pltpu.async_copy(srcs.at[pl.ds(off, gsz)], src_v.at[b], isem)
                pltpu.async_copy(dst2.at[pl.ds(off // ch, kb)],
                                 dst_v.at[b], isem)

        def drain_idx(b):
            pltpu.make_async_copy(srcs.at[pl.ds(0, gsz)],
                                  src_v.at[b], isem).wait()
            pltpu.make_async_copy(dst2.at[pl.ds(0, kb)],
                                  dst_v.at[b], isem).wait()

        def fire_gathers(p, k0, rows, kn, sem):
            return [pltpu.async_copy(
                feat.at[src_v.at[p, pl.ds((k0 + k) * ch, ch)]],
                rows.at[k], sem) for k in range(kn)]

        def drain_rows(rows, kn, sem):
            for k in range(kn):
                pltpu.make_async_copy(zeros.at[pl.ds(0, ch)],
                                      rows.at[k], sem).wait()

        def fire_scatters(p, k0, rows, kn, sem):
            d = [pltpu.async_copy(
                rows.at[k], acc_sh.at[dst_v.at[p, k0 + k]],
                sem, add=True) for k in range(kn)]
            if with_deg:
                d += [pltpu.async_copy(
                    ones_v, deg_sh.at[dst_v.at[p, k0 + k]],
                    sem, add=True) for k in range(kn)]
            return d

        def drain_deg(kn, sem):
            if with_deg:
                for k in range(kn):
                    pltpu.make_async_copy(zeros16.at[pl.ds(0, ch)],
                                          ones_v, sem).wait()

        def steady(g, first):
            p = lax.rem(g, 2)
            w = 1 - p
            if not first:
                drain_rows(rows_b, kbB, ssemB)
                drain_deg(kbB, ssemB)
                fire_idx(lax.rem(g + 1, ng), w, sync=False)
            drain_rows(rows_a, kbA, gsemA)
            sa = fire_scatters(p, 0, rows_a, kbA, ssemA)
            gb = fire_gathers(p, kbA, rows_b, kbB, gsemB)
            for d in gb:
                d.wait()
            fire_scatters(p, kbA, rows_b, kbB, ssemB)
            for d in sa:
                d.wait()
            drain_idx(w)
            fire_gathers(w, 0, rows_a, kbA, gsemA)

        fire_idx(0, 0, sync=True)
        fire_gathers(0, 0, rows_a, kbA, gsemA)
        fire_idx(1, 1, sync=False)
        steady(0, True)

        def group_body(g, carry):
            steady(g, False)
            return carry

        lax.fori_loop(1, ng, group_body, 0)
        drain_rows(rows_b, kbB, ssemB)
        drain_deg(kbB, ssemB)
        drain_rows(rows_a, kbA, gsemA)
        plsc.subcore_barrier()
        pltpu.sync_copy(acc_sh.at[pl.ds(r0, _RPT)],
                        out.at[c, pl.ds(r0, _RPT)])
        if with_deg:
            pltpu.sync_copy(deg_sh.at[pl.ds(r0, _RPT)],
                            degout.at[c, pl.ds(r0, _RPT)])

    return pl.kernel(body,
                     out_type=tuple(out_type) if with_deg else out_type[0],
                     mesh=mesh, scratch_types=scratch,
                     compiler_params=pltpu.CompilerParams(
                         use_tc_tiling_on_sc=False))


_edge_pass_l1 = _make_edge_pass(128, 40, 5, True)
_edge_pass_l2 = _make_edge_pass(32, 80, 25, False)
_edge_pass_l3 = _make_edge_pass(16, 80, 25, False)

_BN = 1024


def _row_spec(w):
    return pl.BlockSpec((_BN, w), lambda i: (i, 0))


def _part_spec(core, rows, w=128):
    return pl.BlockSpec((1, rows, w), lambda i, c=core: (c, i, 0))


def _packed_spec(rows):
    return pl.BlockSpec((rows, 128), lambda i: (i, 0))


def _full_spec(shape):
    return pl.BlockSpec(shape, lambda i: tuple(0 for _ in shape))


def _pack_rows(y, w):
    m = 128 // w
    rows = _BN * w // 128
    ri = lax.broadcasted_iota(jnp.int32, (rows, _BN), 0)
    ni = lax.broadcasted_iota(jnp.int32, (rows, _BN), 1)
    parts = []
    for a in range(m):
        S = (ni == m * ri + a).astype(jnp.float32)
        parts.append(jnp.dot(S, y, preferred_element_type=jnp.float32))
    return jnp.concatenate(parts, axis=1)


def _unpack_rows(pk, w):
    m = 128 // w
    rows = _BN * w // 128
    ni = lax.broadcasted_iota(jnp.int32, (_BN, rows), 0)
    ri = lax.broadcasted_iota(jnp.int32, (_BN, rows), 1)
    A = (ni // m == ri).astype(jnp.float32)
    Y = jnp.dot(A, pk, preferred_element_type=jnp.float32)
    col = lax.broadcasted_iota(jnp.int32, (_BN, 1), 0) % m
    out = jnp.zeros((_BN, w), jnp.float32)
    for a in range(m):
        out = out + Y[:, a * w:(a + 1) * w] * (col == a).astype(jnp.float32)
    return out


def _tc1_body(f0, f1, d0, d1, xr, wl1, wr1, b1, wl2, wr2,
              p2o, r2o, invo):
    degs = _unpack_rows(d0[0] + d1[0], _DEGW)
    inv = 1.0 / jnp.maximum(degs[:, 0:1], 1.0)
    agg = (f0[0] + f1[0]) * inv
    h = jnp.dot(agg, wl1[:], preferred_element_type=jnp.float32)
    h = h + jnp.dot(xr[:], wr1[:], preferred_element_type=jnp.float32)
    h = jnp.maximum(h + b1[:], 0.0)
    p2o[:] = _pack_rows(
        jnp.dot(h, wl2[:], preferred_element_type=jnp.float32), 32)
    r2o[:] = jnp.dot(h, wr2[:], preferred_element_type=jnp.float32)
    invo[:] = jnp.broadcast_to(inv, (_BN, _DEGW))


def _tc1(featp, degpk, x, Wl1, Wr1, b1, Wl2, Wr2):
    n32 = _BN * 32 // 128
    n16 = _BN * 16 // 128
    return pl.pallas_call(
        _tc1_body,
        grid=(_NP // _BN,),
        in_specs=[_part_spec(0, _BN), _part_spec(1, _BN),
                  _part_spec(0, n16), _part_spec(1, n16),
                  _row_spec(128),
                  _full_spec((128, 256)), _full_spec((128, 256)),
                  _full_spec((1, 256)),
                  _full_spec((256, 32)), _full_spec((256, 32))],
        out_specs=[_packed_spec(n32), _row_spec(32), _row_spec(_DEGW)],
        out_shape=[jax.ShapeDtypeStruct((_NP * 32 // 128, 128), jnp.float32),
                   jax.ShapeDtypeStruct((_NP, 32), jnp.float32),
                   jax.ShapeDtypeStruct((_NP, _DEGW), jnp.float32)],
    )(featp, featp, degpk, degpk, x, Wl1, Wr1, b1, Wl2, Wr2)


def _tc2_body(q0, q1, iv, r2, b2, wl3, wr3, p3o, r3o):
    inv = iv[:, 0:1]
    q = _unpack_rows(q0[0] + q1[0], 32)
    h = jnp.maximum(q * inv + b2[:] + r2[:], 0.0)
    p3o[:] = _pack_rows(
        jnp.dot(h, wl3[:], preferred_element_type=jnp.float32), 16)
    r3o[:] = jnp.dot(h, wr3[:], preferred_element_type=jnp.float32)


def _tc2(aggpk, invb, r2, b2, Wl3, Wr3):
    n32 = _BN * 32 // 128
    n16 = _BN * 16 // 128
    return pl.pallas_call(
        _tc2_body,
        grid=(_NP // _BN,),
        in_specs=[_part_spec(0, n32), _part_spec(1, n32),
                  _row_spec(_DEGW), _row_spec(32),
                  _full_spec((1, 32)),
                  _full_spec((32, 16)), _full_spec((32, 16))],
        out_specs=[_packed_spec(n16), _row_spec(16)],
        out_shape=[jax.ShapeDtypeStruct((_NP * 16 // 128, 128), jnp.float32),
                   jax.ShapeDtypeStruct((_NP, 16), jnp.float32)],
    )(aggpk, aggpk, invb, r2, b2, Wl3, Wr3)


def _tc3_body(t0, t1, iv, r3, b3, w1, bb1, w2, bb2, outo):
    inv = iv[:, 0:1]
    t = _unpack_rows(t0[0] + t1[0], 16)
    h = jnp.maximum(t * inv + b3[:] + r3[:], 0.0)
    h = jnp.maximum(jnp.dot(h, w1[:], preferred_element_type=jnp.float32)
                    + bb1[:], 0.0)
    logits = jnp.dot(h, w2[:], preferred_element_type=jnp.float32) + bb2[:]
    m = jnp.max(logits, axis=1, keepdims=True)
    z = logits - m
    lse = jnp.log(jnp.sum(jnp.exp(z), axis=1, keepdims=True))
    outo[:] = z - lse


def _tc3(aggpk, invb, r3, b3, fcW1, fcb1, fcW2, fcb2):
    n16 = _BN * 16 // 128
    return pl.pallas_call(
        _tc3_body,
        grid=(_NP // _BN,),
        in_specs=[_part_spec(0, n16), _part_spec(1, n16),
                  _row_spec(_DEGW), _row_spec(16),
                  _full_spec((1, 16)),
                  _full_spec((16, 8)), _full_spec((1, 8)),
                  _full_spec((8, 2)), _full_spec((1, 2))],
        out_specs=[_row_spec(2)],
        out_shape=[jax.ShapeDtypeStruct((_N, 2), jnp.float32)],
    )(aggpk, aggpk, invb, r3, b3, fcW1, fcb1, fcW2, fcb2)[0]


def kernel(x, edge_index, Wl1, Wr1, b1, Wl2, Wr2, b2, Wl3, Wr3, b3,
           fcW1, fcb1, fcW2, fcb2):
    src = edge_index[0].astype(jnp.int32)
    dst = edge_index[1].astype(jnp.int32)
    dst2_40 = dst.reshape(_E // 40, 40)
    dst2_80 = dst.reshape(_E // 80, 80)
    zeros128 = jnp.zeros((_NP, 128), jnp.float32)
    zeros32 = jnp.zeros((_NP, 32), jnp.float32)
    zeros16f = jnp.zeros((_NP, 16), jnp.float32)

    featp, degp = _edge_pass_l1(x, src, dst2_40, zeros128, zeros16f)
    degpk = degp.reshape(_NC, _NP * _DEGW // 128, 128)
    p2pk, r2, invb = _tc1(featp, degpk, x, Wl1, Wr1,
                          b1.reshape(1, -1), Wl2, Wr2)
    agg2p = _edge_pass_l2(p2pk.reshape(_NP, 32), src, dst2_80, zeros32)
    p3pk, r3 = _tc2(agg2p.reshape(_NC, _NP * 32 // 128, 128), invb, r2,
                    b2.reshape(1, -1), Wl3, Wr3)
    agg3p = _edge_pass_l3(p3pk.reshape(_NP, 16), src, dst2_80, zeros16f)
    return _tc3(agg3p.reshape(_NC, _NP * 16 // 128, 128), invb, r3,
                b3.reshape(1, -1),
                fcW1, fcb1.reshape(1, -1), fcW2, fcb2.reshape(1, -1))


# speedup vs baseline: 19.3570x; 1.0285x over previous
import jax
import jax.numpy as jnp
from jax import lax
from jax.experimental import pallas as pl
from jax.experimental.pallas import tpu as pltpu
from jax.experimental.pallas import tpu_sc as plsc

_N = 10000
_NP = 10240
_E = 320000
_NC = 2
_NS = 16
_NW = _NC * _NS
_EPW = _E // _NW
_RPT = _NP // _NS
_DEGW = 16


def _make_edge_pass(D, ch, kb, with_deg):
    kbA = (kb + 1) // 2
    kbB = kb - kbA
    gsz = kb * ch
    ng = _EPW // gsz
    assert ng * gsz == _EPW and ch % 8 == 0 and ch <= 128 and ng >= 2
    mesh = plsc.VectorSubcoreMesh(
        core_axis_name="c", subcore_axis_name="s",
        num_cores=_NC, num_subcores=_NS)
    scratch = [
        pltpu.VMEM((2, gsz), jnp.int32),
        pltpu.VMEM((2, kb, ch), jnp.int32),
        pltpu.VMEM((kbA, ch, D), jnp.float32),
        pltpu.VMEM((kbB, ch, D), jnp.float32),
        pltpu.VMEM_SHARED((_NP, D), jnp.float32),
        pltpu.SemaphoreType.DMA,
        pltpu.SemaphoreType.DMA,
        pltpu.SemaphoreType.DMA,
        pltpu.SemaphoreType.DMA,
        pltpu.SemaphoreType.DMA,
    ]
    out_type = [jax.ShapeDtypeStruct((_NC, _NP, D), jnp.float32)]
    if with_deg:
        scratch += [
            pltpu.VMEM((ch, _DEGW), jnp.float32),
            pltpu.VMEM_SHARED((_NP, _DEGW), jnp.float32),
        ]
        out_type.append(jax.ShapeDtypeStruct((_NC, _NP, _DEGW), jnp.float32))

    def body(*refs):
        if with_deg:
            (feat, srcs, dst2, zeros, zeros16, out, degout,
             src_v, dst_v, rows_a, rows_b, acc_sh,
             isem, gsemA, gsemB, ssemA, ssemB, ones_v, deg_sh) = refs
        else:
            (feat, srcs, dst2, zeros, out,
             src_v, dst_v, rows_a, rows_b, acc_sh,
             isem, gsemA, gsemB, ssemA, ssemB) = refs
        c = lax.axis_index("c")
        s = lax.axis_index("s")
        wid = c * _NS + s
        r0 = s * _RPT
        pltpu.sync_copy(zeros.at[pl.ds(r0, _RPT)], acc_sh.at[pl.ds(r0, _RPT)])
        if with_deg:
            pltpu.sync_copy(zeros16.at[pl.ds(r0, _RPT)],
                            deg_sh.at[pl.ds(r0, _RPT)])
            pltpu.sync_copy(zeros16.at[pl.ds(0, ch)], ones_v)

            def fill_ones(i, carry):
                ones_v[i] = jnp.ones((_DEGW,), jnp.float32)
                return carry

            lax.fori_loop(0, ch, fill_ones, 0)
        plsc.subcore_barrier()
        e0 = wid * _EPW

        def fire_idx(g, b, sync):
            off = e0 + g * gsz
            if sync:
                pltpu.sync_copy(srcs.at[pl.ds(off, gsz)], src_v.at[b])
                pltpu.sync_copy(dst2.at[pl.ds(off // ch, kb)], dst_v.at[b])
            else:
                pltpu.async_copy(srcs.at[pl.ds(off, gsz)], src_v.at[b], isem)
                pltpu.async_copy(dst2.at[pl.ds(off // ch, kb)],
                                 dst_v.at[b], isem)

        def drain_idx(b):
            pltpu.make_async_copy(srcs.at[pl.ds(0, gsz)],
                                  src_v.at[b], isem).wait()
            pltpu.make_async_copy(dst2.at[pl.ds(0, kb)],
                                  dst_v.at[b], isem).wait()

        def fire_gathers(p, k0, rows, kn, sem):
            return [pltpu.async_copy(
                feat.at[src_v.at[p, pl.ds((k0 + k) * ch, ch)]],
                rows.at[k], sem) for k in range(kn)]

        def drain_rows(rows, kn, sem):
            for k in range(kn):
                pltpu.make_async_copy(zeros.at[pl.ds(0, ch)],
                                      rows.at[k], sem).wait()

        def fire_scatters(p, k0, rows, kn, sem):
            d = [pltpu.async_copy(
                rows.at[k], acc_sh.at[dst_v.at[p, k0 + k]],
                sem, add=True) for k in range(kn)]
            if with_deg:
                d += [pltpu.async_copy(
                    ones_v, deg_sh.at[dst_v.at[p, k0 + k]],
                    sem, add=True) for k in range(kn)]
            return d

        def drain_deg(kn, sem):
            if with_deg:
                for k in range(kn):
                    pltpu.make_async_copy(zeros16.at[pl.ds(0, ch)],
                                          ones_v, sem).wait()

        def steady(g, first):
            p = lax.rem(g, 2)
            w = 1 - p
            if not first:
                drain_rows(rows_b, kbB, ssemB)
                drain_deg(kbB, ssemB)
                fire_idx(lax.rem(g + 1, ng), w, sync=False)
            drain_rows(rows_a, kbA, gsemA)
            sa = fire_scatters(p, 0, rows_a, kbA, ssemA)
            gb = fire_gathers(p, kbA, rows_b, kbB, gsemB)
            for d in gb:
                d.wait()
            fire_scatters(p, kbA, rows_b, kbB, ssemB)
            for d in sa:
                d.wait()
            drain_idx(w)
            fire_gathers(w, 0, rows_a, kbA, gsemA)

        fire_idx(0, 0, sync=True)
        fire_gathers(0, 0, rows_a, kbA, gsemA)
        fire_idx(1, 1, sync=False)
        steady(0, True)

        def group_body(g, carry):
            steady(g, False)
            return carry

        lax.fori_loop(1, ng, group_body, 0)
        drain_rows(rows_b, kbB, ssemB)
        drain_deg(kbB, ssemB)
        drain_rows(rows_a, kbA, gsemA)
        plsc.subcore_barrier()
        pltpu.sync_copy(acc_sh.at[pl.ds(r0, _RPT)],
                        out.at[c, pl.ds(r0, _RPT)])
        if with_deg:
            pltpu.sync_copy(deg_sh.at[pl.ds(r0, _RPT)],
                            degout.at[c, pl.ds(r0, _RPT)])

    return pl.kernel(body,
                     out_type=tuple(out_type) if with_deg else out_type[0],
                     mesh=mesh, scratch_types=scratch,
                     compiler_params=pltpu.CompilerParams(
                         use_tc_tiling_on_sc=False))


_edge_pass_l1 = _make_edge_pass(128, 40, 5, True)
_edge_pass_l2 = _make_edge_pass(32, 80, 25, False)
_edge_pass_l3 = _make_edge_pass(16, 80, 25, False)

_BN = 1024


def _row_spec(w):
    return pl.BlockSpec((_BN, w), lambda i: (i, 0))


def _part_spec(core, rows, w=128):
    return pl.BlockSpec((1, rows, w), lambda i, c=core: (c, i, 0))


def _packed_spec(rows):
    return pl.BlockSpec((rows, 128), lambda i: (i, 0))


def _full_spec(shape):
    return pl.BlockSpec(shape, lambda i: tuple(0 for _ in shape))


def _pack_rows(y, w):
    m = 128 // w
    rows = _BN * w // 128
    y3 = y.reshape(rows, m, w)
    return jnp.concatenate([y3[:, a, :] for a in range(m)], axis=1)


def _unpack_rows(pk, w):
    m = 128 // w
    st = jnp.stack([pk[:, a * w:(a + 1) * w] for a in range(m)], axis=1)
    return st.reshape(_BN, w)


def _tc1_body(f0, f1, d0, d1, xr, wl1, wr1, b1, wl2, wr2,
              p2o, r2o, invo):
    degs = _unpack_rows(d0[0] + d1[0], _DEGW)
    inv = 1.0 / jnp.maximum(degs[:, 0:1], 1.0)
    agg = (f0[0] + f1[0]) * inv
    h = jnp.dot(agg, wl1[:], preferred_element_type=jnp.float32)
    h = h + jnp.dot(xr[:], wr1[:], preferred_element_type=jnp.float32)
    h = jnp.maximum(h + b1[:], 0.0)
    p2o[:] = _pack_rows(
        jnp.dot(h, wl2[:], preferred_element_type=jnp.float32), 32)
    r2o[:] = jnp.dot(h, wr2[:], preferred_element_type=jnp.float32)
    invo[:] = jnp.broadcast_to(inv, (_BN, _DEGW))


def _tc1(featp, degpk, x, Wl1, Wr1, b1, Wl2, Wr2):
    n32 = _BN * 32 // 128
    n16 = _BN * 16 // 128
    return pl.pallas_call(
        _tc1_body,
        grid=(_NP // _BN,),
        in_specs=[_part_spec(0, _BN), _part_spec(1, _BN),
                  _part_spec(0, n16), _part_spec(1, n16),
                  _row_spec(128),
                  _full_spec((128, 256)), _full_spec((128, 256)),
                  _full_spec((1, 256)),
                  _full_spec((256, 32)), _full_spec((256, 32))],
        out_specs=[_packed_spec(n32), _row_spec(32), _row_spec(_DEGW)],
        out_shape=[jax.ShapeDtypeStruct((_NP * 32 // 128, 128), jnp.float32),
                   jax.ShapeDtypeStruct((_NP, 32), jnp.float32),
                   jax.ShapeDtypeStruct((_NP, _DEGW), jnp.float32)],
    )(featp, featp, degpk, degpk, x, Wl1, Wr1, b1, Wl2, Wr2)


def _tc2_body(q0, q1, iv, r2, b2, wl3, wr3, p3o, r3o):
    inv = iv[:, 0:1]
    q = _unpack_rows(q0[0] + q1[0], 32)
    h = jnp.maximum(q * inv + b2[:] + r2[:], 0.0)
    p3o[:] = _pack_rows(
        jnp.dot(h, wl3[:], preferred_element_type=jnp.float32), 16)
    r3o[:] = jnp.dot(h, wr3[:], preferred_element_type=jnp.float32)


def _tc2(aggpk, invb, r2, b2, Wl3, Wr3):
    n32 = _BN * 32 // 128
    n16 = _BN * 16 // 128
    return pl.pallas_call(
        _tc2_body,
        grid=(_NP // _BN,),
        in_specs=[_part_spec(0, n32), _part_spec(1, n32),
                  _row_spec(_DEGW), _row_spec(32),
                  _full_spec((1, 32)),
                  _full_spec((32, 16)), _full_spec((32, 16))],
        out_specs=[_packed_spec(n16), _row_spec(16)],
        out_shape=[jax.ShapeDtypeStruct((_NP * 16 // 128, 128), jnp.float32),
                   jax.ShapeDtypeStruct((_NP, 16), jnp.float32)],
    )(aggpk, aggpk, invb, r2, b2, Wl3, Wr3)


def _tc3_body(t0, t1, iv, r3, b3, w1, bb1, w2, bb2, outo):
    inv = iv[:, 0:1]
    t = _unpack_rows(t0[0] + t1[0], 16)
    h = jnp.maximum(t * inv + b3[:] + r3[:], 0.0)
    h = jnp.maximum(jnp.dot(h, w1[:], preferred_element_type=jnp.float32)
                    + bb1[:], 0.0)
    logits = jnp.dot(h, w2[:], preferred_element_type=jnp.float32) + bb2[:]
    m = jnp.max(logits, axis=1, keepdims=True)
    z = logits - m
    lse = jnp.log(jnp.sum(jnp.exp(z), axis=1, keepdims=True))
    outo[:] = z - lse


def _tc3(aggpk, invb, r3, b3, fcW1, fcb1, fcW2, fcb2):
    n16 = _BN * 16 // 128
    return pl.pallas_call(
        _tc3_body,
        grid=(_NP // _BN,),
        in_specs=[_part_spec(0, n16), _part_spec(1, n16),
                  _row_spec(_DEGW), _row_spec(16),
                  _full_spec((1, 16)),
                  _full_spec((16, 8)), _full_spec((1, 8)),
                  _full_spec((8, 2)), _full_spec((1, 2))],
        out_specs=[_row_spec(2)],
        out_shape=[jax.ShapeDtypeStruct((_N, 2), jnp.float32)],
    )(aggpk, aggpk, invb, r3, b3, fcW1, fcb1, fcW2, fcb2)[0]


def kernel(x, edge_index, Wl1, Wr1, b1, Wl2, Wr2, b2, Wl3, Wr3, b3,
           fcW1, fcb1, fcW2, fcb2):
    src = edge_index[0].astype(jnp.int32)
    dst = edge_index[1].astype(jnp.int32)
    dst2_40 = dst.reshape(_E // 40, 40)
    dst2_80 = dst.reshape(_E // 80, 80)
    zeros128 = jnp.zeros((_NP, 128), jnp.float32)
    zeros32 = jnp.zeros((_NP, 32), jnp.float32)
    zeros16f = jnp.zeros((_NP, 16), jnp.float32)

    featp, degp = _edge_pass_l1(x, src, dst2_40, zeros128, zeros16f)
    degpk = degp.reshape(_NC, _NP * _DEGW // 128, 128)
    p2pk, r2, invb = _tc1(featp, degpk, x, Wl1, Wr1,
                          b1.reshape(1, -1), Wl2, Wr2)
    agg2p = _edge_pass_l2(p2pk.reshape(_NP, 32), src, dst2_80, zeros32)
    p3pk, r3 = _tc2(agg2p.reshape(_NC, _NP * 32 // 128, 128), invb, r2,
                    b2.reshape(1, -1), Wl3, Wr3)
    agg3p = _edge_pass_l3(p3pk.reshape(_NP, 16), src, dst2_80, zeros16f)
    return _tc3(agg3p.reshape(_NC, _NP * 16 // 128, 128), invb, r3,
                b3.reshape(1, -1),
                fcW1, fcb1.reshape(1, -1), fcW2, fcb2.reshape(1, -1))


# speedup vs baseline: 19.9514x; 1.0307x over previous
import jax
import jax.numpy as jnp
from jax import lax
from jax.experimental import pallas as pl
from jax.experimental.pallas import tpu as pltpu
from jax.experimental.pallas import tpu_sc as plsc

_N = 10000
_NP = 10240
_E = 320000
_NC = 2
_NS = 16
_NW = _NC * _NS
_EPW = _E // _NW
_RPT = _NP // _NS
_DEGW = 16


def _make_edge_pass(D, ch, kb, with_deg):
    kbA = (kb + 1) // 2
    kbB = kb - kbA
    gsz = kb * ch
    ng = _EPW // gsz
    assert ng * gsz == _EPW and ch % 8 == 0 and ch <= 128 and ng >= 2
    mesh = plsc.VectorSubcoreMesh(
        core_axis_name="c", subcore_axis_name="s",
        num_cores=_NC, num_subcores=_NS)
    scratch = [
        pltpu.VMEM((2, kb, ch), jnp.int32),
        pltpu.VMEM((2, kb, ch), jnp.int32),
        pltpu.VMEM((kbA, ch, D), jnp.float32),
        pltpu.VMEM((kbB, ch, D), jnp.float32),
        pltpu.VMEM_SHARED((_NP, D), jnp.float32),
        pltpu.SemaphoreType.DMA,
        pltpu.SemaphoreType.DMA,
        pltpu.SemaphoreType.DMA,
        pltpu.SemaphoreType.DMA,
        pltpu.SemaphoreType.DMA,
    ]
    out_type = [jax.ShapeDtypeStruct((_NC, _NP, D), jnp.float32)]
    if with_deg:
        scratch += [
            pltpu.VMEM((ch, _DEGW), jnp.float32),
            pltpu.VMEM_SHARED((_NP, _DEGW), jnp.float32),
        ]
        out_type.append(jax.ShapeDtypeStruct((_NC, _NP, _DEGW), jnp.float32))

    def body(*refs):
        if with_deg:
            (feat, ei2, zeros, zeros16, out, degout,
             src_v, dst_v, rows_a, rows_b, acc_sh,
             isem, gsemA, gsemB, ssemA, ssemB, ones_v, deg_sh) = refs
        else:
            (feat, ei2, zeros, out,
             src_v, dst_v, rows_a, rows_b, acc_sh,
             isem, gsemA, gsemB, ssemA, ssemB) = refs
        c = lax.axis_index("c")
        s = lax.axis_index("s")
        wid = c * _NS + s
        r0 = s * _RPT
        pltpu.sync_copy(zeros.at[pl.ds(r0, _RPT)], acc_sh.at[pl.ds(r0, _RPT)])
        if with_deg:
            pltpu.sync_copy(zeros16.at[pl.ds(r0, _RPT)],
                            deg_sh.at[pl.ds(r0, _RPT)])
            pltpu.sync_copy(zeros16.at[pl.ds(0, ch)], ones_v)

            def fill_ones(i, carry):
                ones_v[i] = jnp.ones((_DEGW,), jnp.float32)
                return carry

            lax.fori_loop(0, ch, fill_ones, 0)
        plsc.subcore_barrier()
        e0 = wid * _EPW

        def fire_idx(g, b, sync):
            off = e0 + g * gsz
            sr = off // ch
            dr = _E // ch + sr
            if sync:
                pltpu.sync_copy(ei2.at[pl.ds(sr, kb)], src_v.at[b])
                pltpu.sync_copy(ei2.at[pl.ds(dr, kb)], dst_v.at[b])
            else:
                pltpu.async_copy(ei2.at[pl.ds(sr, kb)], src_v.at[b], isem)
                pltpu.async_copy(ei2.at[pl.ds(dr, kb)], dst_v.at[b], isem)

        def drain_idx(b):
            pltpu.make_async_copy(ei2.at[pl.ds(0, kb)],
                                  src_v.at[b], isem).wait()
            pltpu.make_async_copy(ei2.at[pl.ds(0, kb)],
                                  dst_v.at[b], isem).wait()

        def fire_gathers(p, k0, rows, kn, sem):
            return [pltpu.async_copy(
                feat.at[src_v.at[p, k0 + k]],
                rows.at[k], sem) for k in range(kn)]

        def drain_rows(rows, kn, sem):
            for k in range(kn):
                pltpu.make_async_copy(zeros.at[pl.ds(0, ch)],
                                      rows.at[k], sem).wait()

        def fire_scatters(p, k0, rows, kn, sem):
            d = [pltpu.async_copy(
                rows.at[k], acc_sh.at[dst_v.at[p, k0 + k]],
                sem, add=True) for k in range(kn)]
            if with_deg:
                d += [pltpu.async_copy(
                    ones_v, deg_sh.at[dst_v.at[p, k0 + k]],
                    sem, add=True) for k in range(kn)]
            return d

        def drain_deg(kn, sem):
            if with_deg:
                for k in range(kn):
                    pltpu.make_async_copy(zeros16.at[pl.ds(0, ch)],
                                          ones_v, sem).wait()

        def steady(g, first):
            p = lax.rem(g, 2)
            w = 1 - p
            if not first:
                drain_rows(rows_b, kbB, ssemB)
                drain_deg(kbB, ssemB)
                fire_idx(lax.rem(g + 1, ng), w, sync=False)
            drain_rows(rows_a, kbA, gsemA)
            sa = fire_scatters(p, 0, rows_a, kbA, ssemA)
            gb = fire_gathers(p, kbA, rows_b, kbB, gsemB)
            for d in gb:
                d.wait()
            fire_scatters(p, kbA, rows_b, kbB, ssemB)
            for d in sa:
                d.wait()
            drain_idx(w)
            fire_gathers(w, 0, rows_a, kbA, gsemA)

        fire_idx(0, 0, sync=True)
        fire_gathers(0, 0, rows_a, kbA, gsemA)
        fire_idx(1, 1, sync=False)
        steady(0, True)

        def group_body(g, carry):
            steady(g, False)
            return carry

        lax.fori_loop(1, ng, group_body, 0)
        drain_rows(rows_b, kbB, ssemB)
        drain_deg(kbB, ssemB)
        drain_rows(rows_a, kbA, gsemA)
        plsc.subcore_barrier()
        pltpu.sync_copy(acc_sh.at[pl.ds(r0, _RPT)],
                        out.at[c, pl.ds(r0, _RPT)])
        if with_deg:
            pltpu.sync_copy(deg_sh.at[pl.ds(r0, _RPT)],
                            degout.at[c, pl.ds(r0, _RPT)])

    return pl.kernel(body,
                     out_type=tuple(out_type) if with_deg else out_type[0],
                     mesh=mesh, scratch_types=scratch,
                     compiler_params=pltpu.CompilerParams(
                         use_tc_tiling_on_sc=False))


_edge_pass_l1 = _make_edge_pass(128, 40, 5, True)
_edge_pass_l2 = _make_edge_pass(32, 80, 25, False)
_edge_pass_l3 = _make_edge_pass(16, 80, 25, False)

_BN = 1024


def _row_spec(w):
    return pl.BlockSpec((_BN, w), lambda i: (i, 0))


def _part_spec(core, rows, w=128):
    return pl.BlockSpec((1, rows, w), lambda i, c=core: (c, i, 0))


def _packed_spec(rows):
    return pl.BlockSpec((rows, 128), lambda i: (i, 0))


def _full_spec(shape):
    return pl.BlockSpec(shape, lambda i: tuple(0 for _ in shape))


def _pack_rows(y, w):
    m = 128 // w
    rows = _BN * w // 128
    y3 = y.reshape(rows, m, w)
    return jnp.concatenate([y3[:, a, :] for a in range(m)], axis=1)


def _unpack_rows(pk, w):
    m = 128 // w
    st = jnp.stack([pk[:, a * w:(a + 1) * w] for a in range(m)], axis=1)
    return st.reshape(_BN, w)


def _tc1_body(f0, f1, d0, d1, xr, wl1, wr1, b1, wl2, wr2,
              p2o, r2o, invo):
    degs = _unpack_rows(d0[0] + d1[0], _DEGW)
    inv = 1.0 / jnp.maximum(degs[:, 0:1], 1.0)
    agg = (f0[0] + f1[0]) * inv
    h = jnp.dot(agg, wl1[:], preferred_element_type=jnp.float32)
    h = h + jnp.dot(xr[:], wr1[:], preferred_element_type=jnp.float32)
    h = jnp.maximum(h + b1[:], 0.0)
    p2o[:] = _pack_rows(
        jnp.dot(h, wl2[:], preferred_element_type=jnp.float32), 32)
    r2o[:] = jnp.dot(h, wr2[:], preferred_element_type=jnp.float32)
    invo[:] = jnp.broadcast_to(inv, (_BN, _DEGW))


def _tc1(featp, degpk, x, Wl1, Wr1, b1, Wl2, Wr2):
    n32 = _BN * 32 // 128
    n16 = _BN * 16 // 128
    return pl.pallas_call(
        _tc1_body,
        grid=(_NP // _BN,),
        in_specs=[_part_spec(0, _BN), _part_spec(1, _BN),
                  _part_spec(0, n16), _part_spec(1, n16),
                  _row_spec(128),
                  _full_spec((128, 256)), _full_spec((128, 256)),
                  _full_spec((1, 256)),
                  _full_spec((256, 32)), _full_spec((256, 32))],
        out_specs=[_packed_spec(n32), _row_spec(32), _row_spec(_DEGW)],
        out_shape=[jax.ShapeDtypeStruct((_NP * 32 // 128, 128), jnp.float32),
                   jax.ShapeDtypeStruct((_NP, 32), jnp.float32),
                   jax.ShapeDtypeStruct((_NP, _DEGW), jnp.float32)],
    )(featp, featp, degpk, degpk, x, Wl1, Wr1, b1, Wl2, Wr2)


def _tc2_body(q0, q1, iv, r2, b2, wl3, wr3, p3o, r3o):
    inv = iv[:, 0:1]
    q = _unpack_rows(q0[0] + q1[0], 32)
    h = jnp.maximum(q * inv + b2[:] + r2[:], 0.0)
    p3o[:] = _pack_rows(
        jnp.dot(h, wl3[:], preferred_element_type=jnp.float32), 16)
    r3o[:] = jnp.dot(h, wr3[:], preferred_element_type=jnp.float32)


def _tc2(aggpk, invb, r2, b2, Wl3, Wr3):
    n32 = _BN * 32 // 128
    n16 = _BN * 16 // 128
    return pl.pallas_call(
        _tc2_body,
        grid=(_NP // _BN,),
        in_specs=[_part_spec(0, n32), _part_spec(1, n32),
                  _row_spec(_DEGW), _row_spec(32),
                  _full_spec((1, 32)),
                  _full_spec((32, 16)), _full_spec((32, 16))],
        out_specs=[_packed_spec(n16), _row_spec(16)],
        out_shape=[jax.ShapeDtypeStruct((_NP * 16 // 128, 128), jnp.float32),
                   jax.ShapeDtypeStruct((_NP, 16), jnp.float32)],
    )(aggpk, aggpk, invb, r2, b2, Wl3, Wr3)


def _tc3_body(t0, t1, iv, r3, b3, w1, bb1, w2, bb2, outo):
    inv = iv[:, 0:1]
    t = _unpack_rows(t0[0] + t1[0], 16)
    h = jnp.maximum(t * inv + b3[:] + r3[:], 0.0)
    h = jnp.maximum(jnp.dot(h, w1[:], preferred_element_type=jnp.float32)
                    + bb1[:], 0.0)
    logits = jnp.dot(h, w2[:], preferred_element_type=jnp.float32) + bb2[:]
    m = jnp.max(logits, axis=1, keepdims=True)
    z = logits - m
    lse = jnp.log(jnp.sum(jnp.exp(z), axis=1, keepdims=True))
    outo[:] = z - lse


def _tc3(aggpk, invb, r3, b3, fcW1, fcb1, fcW2, fcb2):
    n16 = _BN * 16 // 128
    return pl.pallas_call(
        _tc3_body,
        grid=(_NP // _BN,),
        in_specs=[_part_spec(0, n16), _part_spec(1, n16),
                  _row_spec(_DEGW), _row_spec(16),
                  _full_spec((1, 16)),
                  _full_spec((16, 8)), _full_spec((1, 8)),
                  _full_spec((8, 2)), _full_spec((1, 2))],
        out_specs=[_row_spec(2)],
        out_shape=[jax.ShapeDtypeStruct((_N, 2), jnp.float32)],
    )(aggpk, aggpk, invb, r3, b3, fcW1, fcb1, fcW2, fcb2)[0]


def kernel(x, edge_index, Wl1, Wr1, b1, Wl2, Wr2, b2, Wl3, Wr3, b3,
           fcW1, fcb1, fcW2, fcb2):
    ei_lin = edge_index.astype(jnp.int32).reshape(2 * _E)
    ei2_40 = ei_lin.reshape(2 * _E // 40, 40)
    ei2_80 = ei_lin.reshape(2 * _E // 80, 80)
    zeros128 = jnp.zeros((_NP, 128), jnp.float32)
    zeros32 = jnp.zeros((_NP, 32), jnp.float32)
    zeros16f = jnp.zeros((_NP, 16), jnp.float32)

    featp, degp = _edge_pass_l1(x, ei2_40, zeros128, zeros16f)
    degpk = degp.reshape(_NC, _NP * _DEGW // 128, 128)
    p2pk, r2, invb = _tc1(featp, degpk, x, Wl1, Wr1,
                          b1.reshape(1, -1), Wl2, Wr2)
    agg2p = _edge_pass_l2(p2pk.reshape(_NP, 32), ei2_80, zeros32)
    p3pk, r3 = _tc2(agg2p.reshape(_NC, _NP * 32 // 128, 128), invb, r2,
                    b2.reshape(1, -1), Wl3, Wr3)
    agg3p = _edge_pass_l3(p3pk.reshape(_NP, 16), ei2_80, zeros16f)
    return _tc3(agg3p.reshape(_NC, _NP * 16 // 128, 128), invb, r3,
                b3.reshape(1, -1),
                fcW1, fcb1.reshape(1, -1), fcW2, fcb2.reshape(1, -1))
